# baseline (speedup 1.0000x reference)
import functools

import jax
import jax.numpy as jnp
from jax import lax
from jax.experimental import pallas as pl
from jax.experimental.pallas import tpu as pltpu
from jax.experimental.pallas import tpu_sc as plsc

D_MODEL = 64
SUBROW = 128
NSUB = 5
NBUF = 2


@functools.lru_cache(maxsize=None)
def _make(total_rows):
    info = plsc.get_sparse_core_info()
    nc, ns = info.num_cores, info.num_subcores
    nw = nc * ns
    n_subrows = total_rows // SUBROW
    per_w = n_subrows // nw
    n_chunks = per_w // NSUB
    outer = n_chunks // NBUF
    ch_rows = NSUB * SUBROW

    mesh = plsc.VectorSubcoreMesh(core_axis_name="c", subcore_axis_name="s")

    @functools.partial(
        pl.kernel,
        mesh=mesh,
        out_type=jax.ShapeDtypeStruct((total_rows, D_MODEL), jnp.float32),
        scratch_types=[
            pltpu.VMEM((per_w, SUBROW), jnp.int32),
            pltpu.VMEM((NBUF, ch_rows, D_MODEL), jnp.float32),
            pltpu.SemaphoreType.DMA,
        ],
        compiler_params=pltpu.CompilerParams(use_tc_tiling_on_sc=False),
    )
    def gather_kernel(idx_hbm, table_hbm, out_hbm, idx_v, rows_v, sem):
        wid = lax.axis_index("s") * nc + lax.axis_index("c")
        base = wid * per_w
        pltpu.sync_copy(idx_hbm.at[pl.ds(base, per_w)], idx_v)

        def fire(b, chunk):
            for j in range(NSUB):
                pltpu.async_copy(
                    table_hbm.at[idx_v.at[chunk * NSUB + j]],
                    rows_v.at[b, pl.ds(j * SUBROW, SUBROW)], sem)

        def drain(b):
            pltpu.make_async_copy(
                out_hbm.at[pl.ds(0, ch_rows)], rows_v.at[b], sem).wait()

        def write(b, chunk):
            row0 = (base + chunk * NSUB) * SUBROW
            pltpu.sync_copy(rows_v.at[b], out_hbm.at[pl.ds(row0, ch_rows)])

        fire(0, 0)

        def body(i, carry):
            c0 = i * NBUF
            fire(1, c0 + 1)
            drain(0)
            write(0, c0)

            @pl.when(i < outer - 1)
            def _():
                fire(0, c0 + 2)

            drain(1)
            write(1, c0 + 1)
            return carry

        lax.fori_loop(0, outer, body, 0)

    return gather_kernel


def kernel(x, table):
    b, t = x.shape
    total = b * t
    idx = x.reshape(total // SUBROW, SUBROW).astype(jnp.int32)
    out = _make(total)(idx, table)
    return out.reshape(b, t, D_MODEL)
